# Initial kernel scaffold; baseline (speedup 1.0000x reference)
import jax
import jax.numpy as jnp
from jax.experimental import pallas as pl


def kernel(x, edge_index, edge_attr, W1, b1, g1, be1, W2, b2, g2, be2, Wl1, bl1, Wl2, bl2):
    raise NotImplementedError("write your pallas kernel here")


# speedup vs baseline: 1.6216x; 1.6216x over previous
import functools

import jax
import jax.numpy as jnp
from jax.experimental import pallas as pl
from jax.experimental.pallas import tpu as pltpu

_EPS = 1e-5


def _stats1_body(n_rows, x_ref, w1_ref, g1_ref, be1_ref, a1_ref, c1_ref):
    x = x_ref[...]
    w1 = w1_ref[...]
    gram = jax.lax.dot_general(
        x, x, (((0,), (0,)), ((), ())), preferred_element_type=jnp.float32
    )
    mu = jnp.sum(x, axis=0, keepdims=True) / n_rows
    m1 = jnp.dot(mu, w1, preferred_element_type=jnp.float32)
    gw = jnp.dot(gram, w1, preferred_element_type=jnp.float32)
    ex2 = jnp.sum(gw * w1, axis=0, keepdims=True) / n_rows
    var = ex2 - m1 * m1
    inv = jax.lax.rsqrt(var + _EPS)
    a1 = g1_ref[...] * inv
    a1_ref[...] = a1
    c1_ref[...] = be1_ref[...] - m1 * a1


def _mid_body(n_rows, n_tiles, x_ref, w1_ref, w2_ref, a1_ref, c1_ref,
              g2_ref, be2_ref, z2_ref, a2_ref, c2_ref, s2_ref, q2_ref):
    t = pl.program_id(0)
    z1 = jnp.dot(x_ref[...], w1_ref[...], preferred_element_type=jnp.float32)
    h1 = jnp.maximum(z1 * a1_ref[...] + c1_ref[...], 0.0)
    z2 = jnp.dot(h1, w2_ref[...], preferred_element_type=jnp.float32)
    z2_ref[...] = z2
    ps = jnp.sum(z2, axis=0, keepdims=True)
    pq = jnp.sum(z2 * z2, axis=0, keepdims=True)

    @pl.when(t == 0)
    def _init():
        s2_ref[...] = ps
        q2_ref[...] = pq

    @pl.when(t > 0)
    def _acc():
        s2_ref[...] += ps
        q2_ref[...] += pq

    @pl.when(t == n_tiles - 1)
    def _finish():
        m = s2_ref[...] / n_rows
        var = q2_ref[...] / n_rows - m * m
        inv = jax.lax.rsqrt(var + _EPS)
        a2 = g2_ref[...] * inv
        a2_ref[...] = a2
        c2_ref[...] = be2_ref[...] - m * a2


def _tail_body(z2_ref, a2_ref, c2_ref, wl1_ref, bl1_ref, wl2_ref, bl2_ref,
               out_ref):
    h2 = jnp.maximum(z2_ref[...] * a2_ref[...] + c2_ref[...], 0.0)
    t = jnp.maximum(
        jnp.dot(h2, wl1_ref[...], preferred_element_type=jnp.float32)
        + bl1_ref[...], 0.0)
    o = (jnp.dot(t, wl2_ref[...], preferred_element_type=jnp.float32)
         + bl2_ref[...])
    m = jnp.max(o, axis=1, keepdims=True)
    lse = jnp.log(jnp.sum(jnp.exp(o - m), axis=1, keepdims=True)) + m
    out_ref[...] = o - lse


def kernel(x, edge_index, edge_attr, W1, b1, g1, be1, W2, b2, g2, be2,
           Wl1, bl1, Wl2, bl2):
    del edge_index, edge_attr, b1, b2
    n, d = x.shape
    h = W1.shape[1]
    mid = Wl1.shape[1]
    c = Wl2.shape[1]

    g1r = g1.reshape(1, h)
    be1r = be1.reshape(1, h)
    g2r = g2.reshape(1, h)
    be2r = be2.reshape(1, h)
    bl1r = bl1.reshape(1, mid)
    bl2r = bl2.reshape(1, c)

    r = 1
    for cand in (1000, 400, 200, 80, 40, 16, 8):
        if n % cand == 0:
            r = cand
            break
    n_tiles = n // r

    a1, c1 = pl.pallas_call(
        functools.partial(_stats1_body, float(n)),
        out_shape=(
            jax.ShapeDtypeStruct((1, h), jnp.float32),
            jax.ShapeDtypeStruct((1, h), jnp.float32),
        ),
    )(x, W1, g1r, be1r)

    z2, a2, c2 = pl.pallas_call(
        functools.partial(_mid_body, float(n), n_tiles),
        grid=(n_tiles,),
        in_specs=[
            pl.BlockSpec((r, d), lambda t: (t, 0)),
            pl.BlockSpec((d, h), lambda t: (0, 0)),
            pl.BlockSpec((h, h), lambda t: (0, 0)),
            pl.BlockSpec((1, h), lambda t: (0, 0)),
            pl.BlockSpec((1, h), lambda t: (0, 0)),
            pl.BlockSpec((1, h), lambda t: (0, 0)),
            pl.BlockSpec((1, h), lambda t: (0, 0)),
        ],
        out_specs=(
            pl.BlockSpec((r, h), lambda t: (t, 0)),
            pl.BlockSpec((1, h), lambda t: (0, 0)),
            pl.BlockSpec((1, h), lambda t: (0, 0)),
        ),
        out_shape=(
            jax.ShapeDtypeStruct((n, h), jnp.float32),
            jax.ShapeDtypeStruct((1, h), jnp.float32),
            jax.ShapeDtypeStruct((1, h), jnp.float32),
        ),
        scratch_shapes=[
            pltpu.VMEM((1, h), jnp.float32),
            pltpu.VMEM((1, h), jnp.float32),
        ],
    )(x, W1, W2, a1, c1, g2r, be2r)

    out = pl.pallas_call(
        _tail_body,
        grid=(n_tiles,),
        in_specs=[
            pl.BlockSpec((r, h), lambda t: (t, 0)),
            pl.BlockSpec((1, h), lambda t: (0, 0)),
            pl.BlockSpec((1, h), lambda t: (0, 0)),
            pl.BlockSpec((h, mid), lambda t: (0, 0)),
            pl.BlockSpec((1, mid), lambda t: (0, 0)),
            pl.BlockSpec((mid, c), lambda t: (0, 0)),
            pl.BlockSpec((1, c), lambda t: (0, 0)),
        ],
        out_specs=pl.BlockSpec((r, c), lambda t: (t, 0)),
        out_shape=jax.ShapeDtypeStruct((n, c), jnp.float32),
    )(z2, a2, c2, Wl1, bl1r, Wl2, bl2r)

    return out
